# baseline (speedup 1.0000x reference)
import functools

import jax
import jax.numpy as jnp
from jax import lax
from jax.experimental import pallas as pl
from jax.experimental.pallas import tpu as pltpu
from jax.experimental.pallas import tpu_sc as plsc


def _clip01(v):
    return jnp.maximum(jnp.zeros((16,), jnp.int32),
                       jnp.minimum(jnp.ones((16,), jnp.int32), v))


def _gather_kernel(x_hbm, out_hbm, buf, obuf):
    pltpu.sync_copy(x_hbm.at[:, pl.ds(0, 128)], buf)
    i = lax.iota(jnp.int32, 16)
    row = _clip01(i - 2)
    col = _clip01(i) - _clip01(i - 3)
    obuf[...] = plsc.load_gather(buf, [row, col])
    pltpu.sync_copy(obuf.at[pl.ds(0, 6)], out_hbm)


def kernel(x):
    mesh = plsc.VectorSubcoreMesh(
        core_axis_name="c", subcore_axis_name="s",
        num_cores=1, num_subcores=1)
    run = functools.partial(
        pl.kernel,
        mesh=mesh,
        compiler_params=pltpu.CompilerParams(needs_layout_passes=False),
        out_type=jax.ShapeDtypeStruct((6,), jnp.float32),
        scratch_types=[
            pltpu.VMEM((2, 128), jnp.float32),
            pltpu.VMEM((16,), jnp.float32),
        ],
    )(_gather_kernel)
    return run(x).reshape(2, 3)
